# baseline (speedup 1.0000x reference)
import functools

import jax
import jax.numpy as jnp
from jax import lax
from jax.experimental import pallas as pl
from jax.experimental.pallas import tpu as pltpu
from jax.experimental.pallas import tpu_sc as plsc

_D = 128
_C = 64
_L = 16
_NB = 4


@functools.cache
def _build_sc_kernel(n_triples: int):
    info = plsc.get_sparse_core_info()
    nc, ns = info.num_cores, info.num_subcores
    nw = nc * ns
    per_w = n_triples // nw
    assert per_w * nw == n_triples and per_w % (_NB * _C) == 0
    n_chunks = per_w // _C
    n_ring = n_chunks // _NB
    mesh = plsc.VectorSubcoreMesh(core_axis_name="c", subcore_axis_name="s")

    row_buf = pltpu.VMEM((_C, _D), jnp.float32)

    @functools.partial(
        pl.kernel,
        mesh=mesh,
        out_type=jax.ShapeDtypeStruct((n_triples,), jnp.float32),
        compiler_params=pltpu.CompilerParams(needs_layout_passes=False),
        scratch_types=[
            pltpu.VMEM((per_w,), jnp.int32),
            pltpu.VMEM((per_w,), jnp.int32),
            pltpu.VMEM((per_w,), jnp.int32),
            [row_buf] * _NB,
            [row_buf] * _NB,
            pltpu.VMEM((2, _D), jnp.float32),
            pltpu.VMEM((_D,), jnp.float32),
            pltpu.VMEM((_C,), jnp.float32),
            [pltpu.SemaphoreType.DMA] * _NB,
            [pltpu.SemaphoreType.DMA] * _NB,
        ],
    )
    def sc_entail(cl_hbm, cr_hbm, d_hbm, table_hbm, out_hbm,
                  cl_ia, cr_ia, d_ia, s_bufs, d_bufs, bt_v, tbh_v, sc_v,
                  semsA, semsB):
        wid = lax.axis_index("s") * nc + lax.axis_index("c")
        base = wid * per_w

        pltpu.sync_copy(table_hbm.at[pl.ds(0, 2)], bt_v)
        for g in range(_D // _L):
            sl = pl.ds(g * _L, _L)
            tbh_v[sl] = 0.5 * (bt_v[1, sl] - bt_v[0, sl])

        pltpu.sync_copy(cl_hbm.at[pl.ds(base, per_w)], cl_ia)
        pltpu.sync_copy(cr_hbm.at[pl.ds(base, per_w)], cr_ia)
        pltpu.sync_copy(d_hbm.at[pl.ds(base, per_w)], d_ia)

        def fire1(k, ch):
            s = pl.ds(ch * _C, _C)
            pltpu.async_copy(table_hbm.at[cl_ia.at[s]], s_bufs[k], semsA[k])
            pltpu.async_copy(table_hbm.at[d_ia.at[s]], d_bufs[k], semsB[k])

        def drainA(k, ch):
            s = pl.ds(ch * _C, _C)
            pltpu.make_async_copy(
                table_hbm.at[cl_ia.at[s]], s_bufs[k], semsA[k]).wait()

        def fire2(k, ch):
            s = pl.ds(ch * _C, _C)
            pltpu.async_copy(
                table_hbm.at[cr_ia.at[s]], s_bufs[k], semsB[k], add=True)

        def drainB(k, ch):
            s = pl.ds(ch * _C, _C)
            pltpu.make_async_copy(
                table_hbm.at[cr_ia.at[s]], s_bufs[k], semsB[k]).wait()
            pltpu.make_async_copy(
                table_hbm.at[d_ia.at[s]], d_bufs[k], semsB[k]).wait()

        lanes = lax.iota(jnp.int32, _L)
        n_g = _C // _L
        rows_list = [jnp.full((_L,), g * _L, jnp.int32) + lanes
                     for g in range(n_g)]

        def compute(k, ch):
            s_r, d_r = s_bufs[k], d_bufs[k]

            def col_body(c, carry2):
                cols = jnp.bitwise_and(
                    jnp.full((_L,), c, jnp.int32) + lanes, _D - 1)
                tb = plsc.load_gather(tbh_v, [cols])
                new = []
                for g in range(n_g):
                    s = plsc.load_gather(s_r, [rows_list[g], cols])
                    dd = plsc.load_gather(d_r, [rows_list[g], cols])
                    diff = 0.5 * s - dd
                    new.append(carry2[2 * g] + diff * diff)
                    new.append(carry2[2 * g + 1] + s * tb)
                return tuple(new)

            accs = lax.fori_loop(
                0, _D, col_body,
                tuple(jnp.zeros((_L,), jnp.float32) for _ in range(2 * n_g)),
                unroll=4)

            for g in range(n_g):
                accd, acct = accs[2 * g], accs[2 * g + 1]
                x = accd + 1e-12
                i = plsc.bitcast(x, jnp.int32)
                i = jnp.full((_L,), 0x5F3759DF, jnp.int32) - jnp.right_shift(i, 1)
                r = plsc.bitcast(i, jnp.float32)
                for _ in range(3):
                    r = r * (1.5 - 0.5 * x * r * r)
                sc_v[pl.ds(g * _L, _L)] = acct - x * r

            pltpu.sync_copy(sc_v, out_hbm.at[pl.ds(base + ch * _C, _C)])

        fire1(0, 0)
        fire1(1, 1)
        drainA(0, 0)
        fire2(0, 0)
        fire1(2, 2)
        drainA(1, 1)
        fire2(1, 1)
        fire1(3, 3)

        def ring_body(it, carry):
            c0 = _NB * it
            for k in range(_NB):
                ck = c0 + k
                k2 = (k + 2) % _NB
                drainB(k, ck)

                @pl.when(ck + 2 < n_chunks)
                def _():
                    drainA(k2, ck + 2)
                    fire2(k2, ck + 2)

                compute(k, ck)

                @pl.when(ck + _NB < n_chunks)
                def _():
                    fire1(k, ck + _NB)

            return carry

        lax.fori_loop(0, n_ring, ring_body, 0)

    return sc_entail


def kernel(x, table):
    bs, num_axioms, ents = x.shape
    assert ents == 3
    xt = x.reshape(-1, 3).astype(jnp.int32).T
    cl, cr, d = xt[0], xt[1], xt[2]
    scores = _build_sc_kernel(bs * num_axioms)(cl, cr, d, table)
    return scores.reshape(bs, num_axioms)


# speedup vs baseline: 1.0060x; 1.0060x over previous
import functools

import jax
import jax.numpy as jnp
from jax import lax
from jax.experimental import pallas as pl
from jax.experimental.pallas import tpu as pltpu
from jax.experimental.pallas import tpu_sc as plsc

_D = 128
_C = 64
_L = 16
_NB = 4


@functools.cache
def _build_sc_kernel(n_triples: int):
    info = plsc.get_sparse_core_info()
    nc, ns = info.num_cores, info.num_subcores
    nw = nc * ns
    per_w = n_triples // nw
    assert per_w * nw == n_triples and per_w % (_NB * _C) == 0
    n_chunks = per_w // _C
    n_ring = n_chunks // _NB
    mesh = plsc.VectorSubcoreMesh(core_axis_name="c", subcore_axis_name="s")

    row_buf = pltpu.VMEM((_C, _D), jnp.float32)

    @functools.partial(
        pl.kernel,
        mesh=mesh,
        out_type=jax.ShapeDtypeStruct((n_triples,), jnp.float32),
        compiler_params=pltpu.CompilerParams(needs_layout_passes=False),
        scratch_types=[
            pltpu.VMEM((per_w,), jnp.int32),
            pltpu.VMEM((per_w,), jnp.int32),
            pltpu.VMEM((per_w,), jnp.int32),
            [row_buf] * _NB,
            [row_buf] * _NB,
            pltpu.VMEM((2, _D), jnp.float32),
            pltpu.VMEM((_D,), jnp.float32),
            [pltpu.VMEM((_C,), jnp.float32)] * 2,
            [pltpu.SemaphoreType.DMA] * _NB,
            [pltpu.SemaphoreType.DMA] * _NB,
            [pltpu.SemaphoreType.DMA] * 2,
        ],
    )
    def sc_entail(cl_hbm, cr_hbm, d_hbm, table_hbm, out_hbm,
                  cl_ia, cr_ia, d_ia, s_bufs, d_bufs, bt_v, tbh_v, sc_vs,
                  semsA, semsB, sems_out):
        wid = lax.axis_index("s") * nc + lax.axis_index("c")
        base = wid * per_w

        pltpu.sync_copy(table_hbm.at[pl.ds(0, 2)], bt_v)
        for g in range(_D // _L):
            sl = pl.ds(g * _L, _L)
            tbh_v[sl] = 0.5 * (bt_v[1, sl] - bt_v[0, sl])

        pltpu.sync_copy(cl_hbm.at[pl.ds(base, per_w)], cl_ia)
        pltpu.sync_copy(cr_hbm.at[pl.ds(base, per_w)], cr_ia)
        pltpu.sync_copy(d_hbm.at[pl.ds(base, per_w)], d_ia)

        def fire1(k, ch):
            s = pl.ds(ch * _C, _C)
            pltpu.async_copy(table_hbm.at[cl_ia.at[s]], s_bufs[k], semsA[k])
            pltpu.async_copy(table_hbm.at[d_ia.at[s]], d_bufs[k], semsB[k])

        def drainA(k, ch):
            s = pl.ds(ch * _C, _C)
            pltpu.make_async_copy(
                table_hbm.at[cl_ia.at[s]], s_bufs[k], semsA[k]).wait()

        def fire2(k, ch):
            s = pl.ds(ch * _C, _C)
            pltpu.async_copy(
                table_hbm.at[cr_ia.at[s]], s_bufs[k], semsB[k], add=True)

        def drainB(k, ch):
            s = pl.ds(ch * _C, _C)
            pltpu.make_async_copy(
                table_hbm.at[cr_ia.at[s]], s_bufs[k], semsB[k]).wait()
            pltpu.make_async_copy(
                table_hbm.at[d_ia.at[s]], d_bufs[k], semsB[k]).wait()

        lanes = lax.iota(jnp.int32, _L)
        n_g = _C // _L
        rows_list = [jnp.full((_L,), g * _L, jnp.int32) + lanes
                     for g in range(n_g)]

        def compute(k, ch, has_pending):
            s_r, d_r = s_bufs[k], d_bufs[k]
            sc_v = sc_vs[k % 2]
            sem_o = sems_out[k % 2]

            def col_body(c, carry2):
                cols = jnp.bitwise_and(
                    jnp.full((_L,), c, jnp.int32) + lanes, _D - 1)
                tb = plsc.load_gather(tbh_v, [cols])
                new = []
                for g in range(n_g):
                    s = plsc.load_gather(s_r, [rows_list[g], cols])
                    dd = plsc.load_gather(d_r, [rows_list[g], cols])
                    diff = 0.5 * s - dd
                    new.append(carry2[2 * g] + diff * diff)
                    new.append(carry2[2 * g + 1] + s * tb)
                return tuple(new)

            accs = lax.fori_loop(
                0, _D, col_body,
                tuple(jnp.zeros((_L,), jnp.float32) for _ in range(2 * n_g)),
                unroll=4)

            @pl.when(has_pending)
            def _():
                pltpu.make_async_copy(
                    sc_v, out_hbm.at[pl.ds(base + (ch - 2) * _C, _C)],
                    sem_o).wait()

            for g in range(n_g):
                accd, acct = accs[2 * g], accs[2 * g + 1]
                x = accd + 1e-12
                i = plsc.bitcast(x, jnp.int32)
                i = jnp.full((_L,), 0x5F3759DF, jnp.int32) - jnp.right_shift(i, 1)
                r = plsc.bitcast(i, jnp.float32)
                for _ in range(3):
                    r = r * (1.5 - 0.5 * x * r * r)
                sc_v[pl.ds(g * _L, _L)] = acct - x * r

            pltpu.async_copy(
                sc_v, out_hbm.at[pl.ds(base + ch * _C, _C)], sem_o)

        fire1(0, 0)
        fire1(1, 1)
        drainA(0, 0)
        fire2(0, 0)
        fire1(2, 2)
        drainA(1, 1)
        fire2(1, 1)
        fire1(3, 3)

        def ring_body(it, carry):
            c0 = _NB * it
            for k in range(_NB):
                ck = c0 + k
                k2 = (k + 2) % _NB
                drainB(k, ck)

                @pl.when(ck + 2 < n_chunks)
                def _():
                    drainA(k2, ck + 2)
                    fire2(k2, ck + 2)

                compute(k, ck, ck >= 2)

                @pl.when(ck + _NB < n_chunks)
                def _():
                    fire1(k, ck + _NB)

            return carry

        lax.fori_loop(0, n_ring, ring_body, 0)

        pltpu.make_async_copy(
            sc_vs[0], out_hbm.at[pl.ds(base + (n_chunks - 2) * _C, _C)],
            sems_out[0]).wait()
        pltpu.make_async_copy(
            sc_vs[1], out_hbm.at[pl.ds(base + (n_chunks - 1) * _C, _C)],
            sems_out[1]).wait()

    return sc_entail


def kernel(x, table):
    bs, num_axioms, ents = x.shape
    assert ents == 3
    xt = x.reshape(-1, 3).astype(jnp.int32).T
    cl, cr, d = xt[0], xt[1], xt[2]
    scores = _build_sc_kernel(bs * num_axioms)(cl, cr, d, table)
    return scores.reshape(bs, num_axioms)
